# baseline (speedup 1.0000x reference)
import functools

import jax
import jax.numpy as jnp
from jax import lax
from jax.experimental import pallas as pl
from jax.experimental.pallas import tpu as pltpu
from jax.experimental.pallas import tpu_sc as plsc

N = 10000
D = 128
E = 320000
NC = 2
NS = 16
NW = NC * NS
N_P = 10240
ROWS_PT = N_P // NS
CHUNK = 128
CPT = 80
DH = D // 2
QTR = CPT // 5
E_PT = CPT * CHUNK
E_P = NW * E_PT
PAD_IDX = N
BM = 128

_sc_mesh = plsc.VectorSubcoreMesh(core_axis_name="c", subcore_axis_name="s")


def _sc_scatter_body(g_hbm, src_hbm, dst_hbm, zrows_hbm, out_hbm,
                     srcv, dstv, rows, agg_sh, gsem):
    c = lax.axis_index("c")
    s = lax.axis_index("s")
    wid = s * NC + c
    pltpu.sync_copy(zrows_hbm, rows)
    for k in range(ROWS_PT // CHUNK):
        pltpu.sync_copy(rows, agg_sh.at[pl.ds(s * ROWS_PT + k * CHUNK, CHUNK)])
    plsc.subcore_barrier()

    def step(i, carry):
        pltpu.sync_copy(src_hbm.at[wid, i], srcv)
        pltpu.async_copy(g_hbm.at[srcv], rows, gsem).wait()
        pltpu.sync_copy(dst_hbm.at[wid, i], dstv)
        pltpu.sync_copy(rows, agg_sh.at[dstv], add=True)
        return carry

    lax.fori_loop(0, CPT, step, 0)
    plsc.subcore_barrier()
    for k in range(ROWS_PT // CHUNK):
        sl = pl.ds(s * ROWS_PT + k * CHUNK, CHUNK)
        pltpu.sync_copy(agg_sh.at[sl], rows)
        pltpu.sync_copy(rows, out_hbm.at[c, sl])


_sc_scatter = pl.kernel(
    _sc_scatter_body,
    out_type=jax.ShapeDtypeStruct((NC, N_P, D), jnp.float32),
    mesh=_sc_mesh,
    scratch_types=[
        pltpu.VMEM((CHUNK,), jnp.int32),
        pltpu.VMEM((CHUNK,), jnp.int32),
        pltpu.VMEM((CHUNK, D), jnp.float32),
        pltpu.VMEM_SHARED((N_P, D), jnp.float32),
        pltpu.SemaphoreType.DMA,
    ],
)


def _sc_deg_body(dst_hbm, ones_hbm, zrows_hbm, out_hbm, dst_all, onesv, buf, deg_sh):
    c = lax.axis_index("c")
    s = lax.axis_index("s")
    wid = s * NC + c
    pltpu.sync_copy(dst_hbm.at[wid], dst_all)
    pltpu.sync_copy(zrows_hbm, buf)
    for k in range(ROWS_PT // CHUNK):
        pltpu.sync_copy(buf, deg_sh.at[pl.ds(s * ROWS_PT + k * CHUNK, CHUNK)])
    pltpu.sync_copy(ones_hbm, onesv)
    plsc.subcore_barrier()

    def step(i, carry):
        pltpu.sync_copy(onesv, deg_sh.at[dst_all.at[i]], add=True)
        return carry

    lax.fori_loop(0, CPT, step, 0)
    plsc.subcore_barrier()
    for k in range(ROWS_PT // CHUNK):
        sl = pl.ds(s * ROWS_PT + k * CHUNK, CHUNK)
        pltpu.sync_copy(deg_sh.at[sl], buf)
        pltpu.sync_copy(buf, out_hbm.at[c, sl])


_sc_deg = pl.kernel(
    _sc_deg_body,
    out_type=jax.ShapeDtypeStruct((NC, N_P, D), jnp.float32),
    mesh=_sc_mesh,
    scratch_types=[
        pltpu.VMEM((CPT, CHUNK), jnp.int32),
        pltpu.VMEM((CHUNK, D), jnp.float32),
        pltpu.VMEM((CHUNK, D), jnp.float32),
        pltpu.VMEM_SHARED((N_P, D), jnp.float32),
    ],
)


def _dis_block(da_ref, db_ref):
    deg = da_ref[:, 0:1] + db_ref[:, 0:1] + 1.0
    return lax.rsqrt(deg)


def _mm(a, w_ref):
    return lax.dot_general(a, w_ref[...], (((1,), (0,)), ((), ())),
                           precision=lax.Precision.HIGHEST,
                           preferred_element_type=jnp.float32)


def _t0_body(x_ref, w_ref, da_ref, db_ref, g_ref):
    g_ref[...] = _dis_block(da_ref, db_ref) * _mm(x_ref[...], w_ref)


def _tmid_body(aa_ref, ab_ref, g_ref, da_ref, db_ref, b_ref, w_ref, gn_ref,
               *, relu):
    dis = _dis_block(da_ref, db_ref)
    o = dis * (aa_ref[...] + ab_ref[...] + g_ref[...]) + b_ref[...]
    if relu:
        o = jnp.maximum(o, 0.0)
    gn_ref[...] = dis * _mm(o, w_ref)


def _t3_body(aa_ref, ab_ref, g_ref, da_ref, db_ref, b_ref, out_ref):
    dis = _dis_block(da_ref, db_ref)
    out_ref[...] = dis * (aa_ref[...] + ab_ref[...] + g_ref[...]) + b_ref[...]


_row_spec = pl.BlockSpec((BM, D), lambda i: (i, 0))
_w_spec = pl.BlockSpec((D, D), lambda i: (0, 0))
_deg_spec = pl.BlockSpec((BM, D), lambda i: (i, 0))
_b_spec = pl.BlockSpec((1, D), lambda i: (0, 0))
_row_out = jax.ShapeDtypeStruct((N_P, D), jnp.float32)

_t0 = pl.pallas_call(
    _t0_body, grid=(N_P // BM,),
    in_specs=[_row_spec, _w_spec, _deg_spec, _deg_spec],
    out_specs=_row_spec, out_shape=_row_out)

_t1 = pl.pallas_call(
    functools.partial(_tmid_body, relu=True), grid=(N_P // BM,),
    in_specs=[_row_spec, _row_spec, _row_spec, _deg_spec, _deg_spec,
              _b_spec, _w_spec],
    out_specs=_row_spec, out_shape=_row_out)

_t2 = pl.pallas_call(
    functools.partial(_tmid_body, relu=False), grid=(N_P // BM,),
    in_specs=[_row_spec, _row_spec, _row_spec, _deg_spec, _deg_spec,
              _b_spec, _w_spec],
    out_specs=_row_spec, out_shape=_row_out)

_t3 = pl.pallas_call(
    _t3_body, grid=(N_P // BM,),
    in_specs=[_row_spec, _row_spec, _row_spec, _deg_spec, _deg_spec, _b_spec],
    out_specs=_row_spec, out_shape=_row_out)


def kernel(x, edge_index, W0, b0, W1, b1, W2, b2):
    src = edge_index[0].astype(jnp.int32)
    dst = edge_index[1].astype(jnp.int32)
    pad = jnp.full((E_P - E,), PAD_IDX, jnp.int32)
    src_p = jnp.concatenate([src, pad]).reshape(NW, CPT, CHUNK)
    dst_p = jnp.concatenate([dst, pad]).reshape(NW, CPT, CHUNK)
    x_p = jnp.pad(x, ((0, N_P - N), (0, 0)))
    zrows = jnp.zeros((CHUNK, D), jnp.float32)
    onerows = jnp.ones((CHUNK, D), jnp.float32)

    degp = _sc_deg(dst_p, onerows, zrows)
    da, db = degp[0], degp[1]
    g0 = _t0(x_p, W0, da, db)
    agg0 = _sc_scatter(g0, src_p, dst_p, zrows)
    g1 = _t1(agg0[0], agg0[1], g0, da, db, b0.reshape(1, D), W1)
    agg1 = _sc_scatter(g1, src_p, dst_p, zrows)
    g2 = _t2(agg1[0], agg1[1], g1, da, db, b1.reshape(1, D), W2)
    agg2 = _sc_scatter(g2, src_p, dst_p, zrows)
    out = _t3(agg2[0], agg2[1], g2, da, db, b2.reshape(1, D))
    return out[:N]


# speedup vs baseline: 1.3107x; 1.3107x over previous
import functools

import jax
import jax.numpy as jnp
from jax import lax
from jax.experimental import pallas as pl
from jax.experimental.pallas import tpu as pltpu
from jax.experimental.pallas import tpu_sc as plsc

N = 10000
D = 128
E = 320000
NC = 2
NS = 16
NW = NC * NS
N_P = 10240
ROWS_PT = N_P // NS
CHUNK = 128
CPT = 80
DH = D // 2
QTR = CPT // 5
E_PT = CPT * CHUNK
E_P = NW * E_PT
CPTS = 79
E_PTS = CPTS * CHUNK
E_PS = NW * E_PTS
PAD_IDX = N
BM = 128

_sc_mesh = plsc.VectorSubcoreMesh(core_axis_name="c", subcore_axis_name="s")


def _sc_scatter_body(g_hbm, src_hbm, dst_hbm, zrows_hbm, out_hbm,
                     srcv, dstv, rows, agg_sh, gsem):
    c = lax.axis_index("c")
    s = lax.axis_index("s")
    wid = s * NC + c
    pltpu.sync_copy(zrows_hbm, rows)
    for k in range(ROWS_PT // CHUNK):
        pltpu.sync_copy(rows, agg_sh.at[pl.ds(s * ROWS_PT + k * CHUNK, CHUNK)])
    plsc.subcore_barrier()
    ebase = wid * E_PTS

    def step(i, carry):
        base = ebase + i * CHUNK
        pltpu.sync_copy(src_hbm.at[pl.ds(base, CHUNK)], srcv)
        pltpu.async_copy(g_hbm.at[srcv], rows, gsem).wait()
        pltpu.sync_copy(dst_hbm.at[pl.ds(base, CHUNK)], dstv)
        pltpu.sync_copy(rows, agg_sh.at[dstv], add=True)
        return carry

    lax.fori_loop(0, CPTS, step, 0)
    plsc.subcore_barrier()
    for k in range(ROWS_PT // CHUNK):
        sl = pl.ds(s * ROWS_PT + k * CHUNK, CHUNK)
        pltpu.sync_copy(agg_sh.at[sl], rows)
        pltpu.sync_copy(rows, out_hbm.at[c, sl])


_sc_scatter = pl.kernel(
    _sc_scatter_body,
    out_type=jax.ShapeDtypeStruct((NC, N_P, D), jnp.float32),
    mesh=_sc_mesh,
    scratch_types=[
        pltpu.VMEM((CHUNK,), jnp.int32),
        pltpu.VMEM((CHUNK,), jnp.int32),
        pltpu.VMEM((CHUNK, D), jnp.float32),
        pltpu.VMEM_SHARED((N_P, D), jnp.float32),
        pltpu.SemaphoreType.DMA,
    ],
)


def _sc_deg_body(dst_hbm, ones_hbm, zrows_hbm, out_hbm, dst_all, onesv, buf, deg_sh):
    c = lax.axis_index("c")
    s = lax.axis_index("s")
    wid = s * NC + c
    pltpu.sync_copy(dst_hbm.at[wid], dst_all)
    pltpu.sync_copy(zrows_hbm, buf)
    for k in range(ROWS_PT // CHUNK):
        pltpu.sync_copy(buf, deg_sh.at[pl.ds(s * ROWS_PT + k * CHUNK, CHUNK)])
    pltpu.sync_copy(ones_hbm, onesv)
    plsc.subcore_barrier()

    def step(i, carry):
        pltpu.sync_copy(onesv, deg_sh.at[dst_all.at[i]], add=True)
        return carry

    lax.fori_loop(0, CPT, step, 0)
    plsc.subcore_barrier()
    for k in range(ROWS_PT // CHUNK):
        sl = pl.ds(s * ROWS_PT + k * CHUNK, CHUNK)
        pltpu.sync_copy(deg_sh.at[sl], buf)
        pltpu.sync_copy(buf, out_hbm.at[c, sl])


_sc_deg = pl.kernel(
    _sc_deg_body,
    out_type=jax.ShapeDtypeStruct((NC, N_P, D), jnp.float32),
    mesh=_sc_mesh,
    scratch_types=[
        pltpu.VMEM((CPT, CHUNK), jnp.int32),
        pltpu.VMEM((CHUNK, D), jnp.float32),
        pltpu.VMEM((CHUNK, D), jnp.float32),
        pltpu.VMEM_SHARED((N_P, D), jnp.float32),
    ],
)


def _dis_block(da_ref, db_ref):
    deg = da_ref[:, 0:1] + db_ref[:, 0:1] + 1.0
    return lax.rsqrt(deg)


def _mm(a, w_ref):
    return lax.dot_general(a, w_ref[...], (((1,), (0,)), ((), ())),
                           precision=lax.Precision.HIGHEST,
                           preferred_element_type=jnp.float32)


def _t0_body(x_ref, w_ref, da_ref, db_ref, g_ref):
    g_ref[...] = _dis_block(da_ref, db_ref) * _mm(x_ref[...], w_ref)


def _tmid_body(aa_ref, ab_ref, g_ref, da_ref, db_ref, b_ref, w_ref, gn_ref,
               *, relu):
    dis = _dis_block(da_ref, db_ref)
    o = dis * (aa_ref[...] + ab_ref[...] + g_ref[...]) + b_ref[...]
    if relu:
        o = jnp.maximum(o, 0.0)
    gn_ref[...] = dis * _mm(o, w_ref)


def _t3_body(aa_ref, ab_ref, g_ref, da_ref, db_ref, b_ref, out_ref):
    dis = _dis_block(da_ref, db_ref)
    out_ref[...] = dis * (aa_ref[...] + ab_ref[...] + g_ref[...]) + b_ref[...]


_row_spec = pl.BlockSpec((BM, D), lambda i: (i, 0))
_w_spec = pl.BlockSpec((D, D), lambda i: (0, 0))
_deg_spec = pl.BlockSpec((BM, D), lambda i: (i, 0))
_b_spec = pl.BlockSpec((1, D), lambda i: (0, 0))
_row_out = jax.ShapeDtypeStruct((N_P, D), jnp.float32)

_t0 = pl.pallas_call(
    _t0_body, grid=(N_P // BM,),
    in_specs=[_row_spec, _w_spec, _deg_spec, _deg_spec],
    out_specs=_row_spec, out_shape=_row_out)

_t1 = pl.pallas_call(
    functools.partial(_tmid_body, relu=True), grid=(N_P // BM,),
    in_specs=[_row_spec, _row_spec, _row_spec, _deg_spec, _deg_spec,
              _b_spec, _w_spec],
    out_specs=_row_spec, out_shape=_row_out)

_t2 = pl.pallas_call(
    functools.partial(_tmid_body, relu=False), grid=(N_P // BM,),
    in_specs=[_row_spec, _row_spec, _row_spec, _deg_spec, _deg_spec,
              _b_spec, _w_spec],
    out_specs=_row_spec, out_shape=_row_out)

_t3 = pl.pallas_call(
    _t3_body, grid=(N_P // BM,),
    in_specs=[_row_spec, _row_spec, _row_spec, _deg_spec, _deg_spec, _b_spec],
    out_specs=_row_spec, out_shape=_row_out)


def kernel(x, edge_index, W0, b0, W1, b1, W2, b2):
    src = edge_index[0].astype(jnp.int32)
    dst = edge_index[1].astype(jnp.int32)
    pad = jnp.full((E_PS - E,), PAD_IDX, jnp.int32)
    src_p = jnp.concatenate([src, pad])
    dst_p = jnp.concatenate([dst, pad])
    pad3 = jnp.full((E_P - E,), PAD_IDX, jnp.int32)
    dst3 = jnp.concatenate([dst, pad3]).reshape(NW, CPT, CHUNK)
    x_p = jnp.pad(x, ((0, N_P - N), (0, 0)))
    zrows = jnp.zeros((CHUNK, D), jnp.float32)
    onerows = jnp.ones((CHUNK, D), jnp.float32)

    degp = _sc_deg(dst3, onerows, zrows)
    da, db = degp[0], degp[1]
    g0 = _t0(x_p, W0, da, db)
    agg0 = _sc_scatter(g0, src_p, dst_p, zrows)
    g1 = _t1(agg0[0], agg0[1], g0, da, db, b0.reshape(1, D), W1)
    agg1 = _sc_scatter(g1, src_p, dst_p, zrows)
    g2 = _t2(agg1[0], agg1[1], g1, da, db, b1.reshape(1, D), W2)
    agg2 = _sc_scatter(g2, src_p, dst_p, zrows)
    out = _t3(agg2[0], agg2[1], g2, da, db, b2.reshape(1, D))
    return out[:N]
